# Initial kernel scaffold; baseline (speedup 1.0000x reference)
import jax
import jax.numpy as jnp
from jax.experimental import pallas as pl


def kernel(x, edge_index, W1, b1, W2, b2):
    raise NotImplementedError("write your pallas kernel here")


# speedup vs baseline: 16.2996x; 16.2996x over previous
import functools

import jax
import jax.numpy as jnp
from jax import lax
from jax.experimental import pallas as pl
from jax.experimental.pallas import tpu as pltpu
from jax.experimental.pallas import tpu_sc as plsc

N = 10000
E = 320000
IN_DIM = 128
HID_DIM = 128
OUT_DIM = 64

NC = 2
NS = 16
NW = NC * NS
EPW = E // NW
CH = 128
NFULL = EPW // CH
TAIL = EPW - NFULL * CH
RSTRIPE = 624
RTAIL = N - NS * RSTRIPE
DEG_MINOR = 8

ROWS_BLK = 400


def _sc_mesh():
    return plsc.VectorSubcoreMesh(core_axis_name="c", subcore_axis_name="s")


def _make_deg_kernel():
    @functools.partial(
        pl.kernel,
        out_type=jax.ShapeDtypeStruct((NC, N, DEG_MINOR), jnp.float32),
        mesh=_sc_mesh(),
        scratch_types=[
            pltpu.VMEM_SHARED((N, DEG_MINOR), jnp.float32),
            pltpu.VMEM((CH,), jnp.int32),
            pltpu.VMEM((TAIL,), jnp.int32),
            pltpu.VMEM((CH, DEG_MINOR), jnp.float32),
        ],
    )
    def deg_kernel(dst_hbm, z_hbm, ones_hbm, out_hbm, acc, dbuf, dbuf_t, ones_v):
        c = lax.axis_index("c")
        s = lax.axis_index("s")
        wid = c * NS + s
        pltpu.sync_copy(z_hbm.at[pl.ds(s * RSTRIPE, RSTRIPE), :],
                        acc.at[pl.ds(s * RSTRIPE, RSTRIPE), :])

        @pl.when(s == NS - 1)
        def _():
            pltpu.sync_copy(z_hbm.at[pl.ds(NS * RSTRIPE, RTAIL), :],
                            acc.at[pl.ds(NS * RSTRIPE, RTAIL), :])

        pltpu.sync_copy(ones_hbm, ones_v)
        plsc.subcore_barrier()
        base = wid * EPW

        def body(i, carry):
            off = base + i * CH
            pltpu.sync_copy(dst_hbm.at[pl.ds(off, CH)], dbuf)
            pltpu.sync_copy(ones_v, acc.at[dbuf], add=True)
            return carry

        lax.fori_loop(0, NFULL, body, 0)
        off = base + NFULL * CH
        pltpu.sync_copy(dst_hbm.at[pl.ds(off, TAIL)], dbuf_t)
        pltpu.sync_copy(ones_v.at[pl.ds(0, TAIL), :], acc.at[dbuf_t], add=True)
        plsc.subcore_barrier()
        pltpu.sync_copy(acc.at[pl.ds(s * RSTRIPE, RSTRIPE), :],
                        out_hbm.at[c, pl.ds(s * RSTRIPE, RSTRIPE), :])

        @pl.when(s == NS - 1)
        def _():
            pltpu.sync_copy(acc.at[pl.ds(NS * RSTRIPE, RTAIL), :],
                            out_hbm.at[c, pl.ds(NS * RSTRIPE, RTAIL), :])

    return deg_kernel


def _make_scatter_kernel(d):
    @functools.partial(
        pl.kernel,
        out_type=jax.ShapeDtypeStruct((NC, N, d), jnp.float32),
        mesh=_sc_mesh(),
        compiler_params=pltpu.CompilerParams(use_tc_tiling_on_sc=False),
        scratch_types=[
            pltpu.VMEM_SHARED((N, d), jnp.float32),
            pltpu.VMEM((CH,), jnp.int32),
            pltpu.VMEM((CH,), jnp.int32),
            pltpu.VMEM((CH, d), jnp.float32),
            pltpu.VMEM((TAIL,), jnp.int32),
            pltpu.VMEM((TAIL,), jnp.int32),
            pltpu.VMEM((TAIL, d), jnp.float32),
            pltpu.SemaphoreType.DMA,
        ],
    )
    def scatter_kernel(g_hbm, src_hbm, dst_hbm, z_hbm, out_hbm,
                       acc, sbuf, dbuf, rows, sbuf_t, dbuf_t, rows_t, sem):
        c = lax.axis_index("c")
        s = lax.axis_index("s")
        wid = c * NS + s
        pltpu.sync_copy(z_hbm.at[pl.ds(s * RSTRIPE, RSTRIPE), :],
                        acc.at[pl.ds(s * RSTRIPE, RSTRIPE), :])

        @pl.when(s == NS - 1)
        def _():
            pltpu.sync_copy(z_hbm.at[pl.ds(NS * RSTRIPE, RTAIL), :],
                            acc.at[pl.ds(NS * RSTRIPE, RTAIL), :])

        plsc.subcore_barrier()
        base = wid * EPW

        def body(i, carry):
            off = base + i * CH
            pltpu.sync_copy(src_hbm.at[pl.ds(off, CH)], sbuf)
            pltpu.sync_copy(dst_hbm.at[pl.ds(off, CH)], dbuf)
            pltpu.async_copy(g_hbm.at[sbuf], rows, sem).wait()
            pltpu.sync_copy(rows, acc.at[dbuf], add=True)
            return carry

        lax.fori_loop(0, NFULL, body, 0)
        off = base + NFULL * CH
        pltpu.sync_copy(src_hbm.at[pl.ds(off, TAIL)], sbuf_t)
        pltpu.sync_copy(dst_hbm.at[pl.ds(off, TAIL)], dbuf_t)
        pltpu.async_copy(g_hbm.at[sbuf_t], rows_t, sem).wait()
        pltpu.sync_copy(rows_t, acc.at[dbuf_t], add=True)
        plsc.subcore_barrier()
        pltpu.sync_copy(acc.at[pl.ds(s * RSTRIPE, RSTRIPE), :],
                        out_hbm.at[c, pl.ds(s * RSTRIPE, RSTRIPE), :])

        @pl.when(s == NS - 1)
        def _():
            pltpu.sync_copy(acc.at[pl.ds(NS * RSTRIPE, RTAIL), :],
                            out_hbm.at[c, pl.ds(NS * RSTRIPE, RTAIL), :])

    return scatter_kernel


_deg_call = _make_deg_kernel()
_scatter_hid = _make_scatter_kernel(HID_DIM)
_scatter_out = _make_scatter_kernel(OUT_DIM)


def _mm1_body(x_ref, w_ref, o_ref):
    o_ref[...] = jnp.dot(x_ref[...], w_ref[...], preferred_element_type=jnp.float32)


def _mm1(x, W1):
    grid = N // ROWS_BLK
    return pl.pallas_call(
        _mm1_body,
        grid=(grid,),
        in_specs=[
            pl.BlockSpec((ROWS_BLK, IN_DIM), lambda i: (i, 0)),
            pl.BlockSpec((IN_DIM, HID_DIM), lambda i: (0, 0)),
        ],
        out_specs=pl.BlockSpec((ROWS_BLK, HID_DIM), lambda i: (i, 0)),
        out_shape=jax.ShapeDtypeStruct((N, HID_DIM), jnp.float32),
    )(x, W1)


def _scale_body(deg_ref, m_ref, dinv_ref, g_ref):
    dsum = jnp.sum(deg_ref[...], axis=0)
    deg = dsum[:, 0:1] + 1.0
    dinv = lax.rsqrt(deg)
    dinv_ref[...] = dinv
    g_ref[...] = m_ref[...] * dinv


def _scale(degp, m1):
    grid = N // ROWS_BLK
    return pl.pallas_call(
        _scale_body,
        grid=(grid,),
        in_specs=[
            pl.BlockSpec((NC, ROWS_BLK, DEG_MINOR), lambda i: (0, i, 0)),
            pl.BlockSpec((ROWS_BLK, HID_DIM), lambda i: (i, 0)),
        ],
        out_specs=[
            pl.BlockSpec((ROWS_BLK, 1), lambda i: (i, 0)),
            pl.BlockSpec((ROWS_BLK, HID_DIM), lambda i: (i, 0)),
        ],
        out_shape=[
            jax.ShapeDtypeStruct((N, 1), jnp.float32),
            jax.ShapeDtypeStruct((N, HID_DIM), jnp.float32),
        ],
    )(degp, m1)


def _combine_mm_body(p_ref, g_ref, dinv_ref, b_ref, w_ref, o_ref):
    dinv = dinv_ref[...]
    s = p_ref[0] + p_ref[1] + g_ref[...]
    h = jnp.maximum(dinv * s + b_ref[...], 0.0)
    m2 = jnp.dot(h, w_ref[...], preferred_element_type=jnp.float32)
    o_ref[...] = m2 * dinv


def _combine_mm(p1, g1, dinv, b1, W2):
    grid = N // ROWS_BLK
    return pl.pallas_call(
        _combine_mm_body,
        grid=(grid,),
        in_specs=[
            pl.BlockSpec((NC, ROWS_BLK, HID_DIM), lambda i: (0, i, 0)),
            pl.BlockSpec((ROWS_BLK, HID_DIM), lambda i: (i, 0)),
            pl.BlockSpec((ROWS_BLK, 1), lambda i: (i, 0)),
            pl.BlockSpec((1, HID_DIM), lambda i: (0, 0)),
            pl.BlockSpec((HID_DIM, OUT_DIM), lambda i: (0, 0)),
        ],
        out_specs=pl.BlockSpec((ROWS_BLK, OUT_DIM), lambda i: (i, 0)),
        out_shape=jax.ShapeDtypeStruct((N, OUT_DIM), jnp.float32),
    )(p1, g1, dinv, b1, W2)


def _final_body(p_ref, g_ref, dinv_ref, b_ref, o_ref):
    z = dinv_ref[...] * (p_ref[0] + p_ref[1] + g_ref[...]) + b_ref[...]
    zmax = jnp.max(z, axis=1, keepdims=True)
    lse = jnp.log(jnp.sum(jnp.exp(z - zmax), axis=1, keepdims=True))
    o_ref[...] = z - zmax - lse


def _final(p2, g2, dinv, b2):
    grid = N // ROWS_BLK
    return pl.pallas_call(
        _final_body,
        grid=(grid,),
        in_specs=[
            pl.BlockSpec((NC, ROWS_BLK, OUT_DIM), lambda i: (0, i, 0)),
            pl.BlockSpec((ROWS_BLK, OUT_DIM), lambda i: (i, 0)),
            pl.BlockSpec((ROWS_BLK, 1), lambda i: (i, 0)),
            pl.BlockSpec((1, OUT_DIM), lambda i: (0, 0)),
        ],
        out_specs=pl.BlockSpec((ROWS_BLK, OUT_DIM), lambda i: (i, 0)),
        out_shape=jax.ShapeDtypeStruct((N, OUT_DIM), jnp.float32),
    )(p2, g2, dinv, b2)


def kernel(x, edge_index, W1, b1, W2, b2):
    ei = edge_index.astype(jnp.int32)
    src = ei[0]
    dst = ei[1]
    z8 = jnp.zeros((N, DEG_MINOR), jnp.float32)
    z128 = jnp.zeros((N, HID_DIM), jnp.float32)
    z64 = jnp.zeros((N, OUT_DIM), jnp.float32)
    ones8 = jnp.ones((CH, DEG_MINOR), jnp.float32)
    b1r = b1.reshape(1, HID_DIM)
    b2r = b2.reshape(1, OUT_DIM)

    degp = _deg_call(dst, z8, ones8)
    m1 = _mm1(x, W1)
    dinv, g1 = _scale(degp, m1)
    p1 = _scatter_hid(g1, src, dst, z128)
    g2 = _combine_mm(p1, g1, dinv, b1r, W2)
    p2 = _scatter_out(g2, src, dst, z64)
    return _final(p2, g2, dinv, b2r)
